# baseline (speedup 1.0000x reference)
import functools

import jax
import jax.numpy as jnp
from jax import lax
from jax.experimental import pallas as pl
from jax.experimental.pallas import tpu as pltpu
from jax.experimental.pallas import tpu_sc as plsc

NC = 2
NS = 16
NW = NC * NS
CHUNK = 128


def _make_sc_kernel(B, D, V, num_cores=NC):
    nw = num_cores * NS
    b_per_w = B // nw
    n_bchunk = b_per_w // CHUNK
    n_chunks = n_bchunk * 3
    mesh = plsc.VectorSubcoreMesh(core_axis_name="c", subcore_axis_name="s",
                                  num_cores=num_cores)

    @functools.partial(
        pl.kernel,
        mesh=mesh,
        compiler_params=pltpu.CompilerParams(needs_layout_passes=False),
        out_type=jax.ShapeDtypeStruct((B,), jnp.float32),
        scratch_types=[
            pltpu.VMEM((n_chunks, CHUNK), jnp.int32),
            pltpu.VMEM((CHUNK, D), jnp.float32),
            pltpu.VMEM((CHUNK, D), jnp.float32),
            pltpu.VMEM((CHUNK, D), jnp.float32),
            pltpu.VMEM((CHUNK, D), jnp.float32),
            pltpu.VMEM((CHUNK, D), jnp.float32),
            pltpu.VMEM((CHUNK, D), jnp.float32),
            pltpu.VMEM((b_per_w,), jnp.float32),
            pltpu.VMEM((3 * D,), jnp.float32),
            pltpu.VMEM((16,), jnp.float32),
            pltpu.SemaphoreType.DMA,
            pltpu.SemaphoreType.DMA,
            pltpu.SemaphoreType.DMA,
            pltpu.SemaphoreType.DMA,
            pltpu.SemaphoreType.DMA,
            pltpu.SemaphoreType.DMA,
        ],
    )
    def sck(w_hbm, b_hbm, idx_hbm, t1_hbm, t2_hbm, out_hbm,
            idx_v, rows0, rows1, rows2, rows3, rows4, rows5,
            out_v, w_v, b_v, sem0, sem1, sem2, sem3, sem4, sem5):
        cid = lax.axis_index("c")
        sid = lax.axis_index("s")
        wid = cid * NS + sid
        base = wid * b_per_w
        pltpu.sync_copy(idx_hbm.at[wid], idx_v)
        pltpu.sync_copy(w_hbm, w_v)
        pltpu.sync_copy(b_hbm, b_v)

        bufs = (rows0, rows1, rows2, rows3, rows4, rows5)
        sems = (sem0, sem1, sem2, sem3, sem4, sem5)
        nbuf = len(bufs)
        tables = (t1_hbm, t1_hbm, t2_hbm)
        iota16 = jnp.arange(16, dtype=jnp.int32)
        lane_masks = [iota16 == i for i in range(16)]
        bvec = b_v[...]

        copies = [None] * n_chunks
        for t in range(min(nbuf, n_chunks)):
            copies[t] = pltpu.async_copy(
                tables[t % 3].at[idx_v.at[t]], bufs[t], sems[t])
        for t in range(n_chunks):
            c, k = t // 3, t % 3
            copies[t].wait()
            rows = bufs[t % nbuf]
            woff = k * D
            wv = [w_v[pl.ds(woff + 16 * kk, 16)] for kk in range(D // 16)]

            def group_body(g, _, _rows=rows, _wv=wv, _c=c, _k=k):
                base_row = g * 16
                zero = jnp.zeros((16,), jnp.float32)
                terms = []
                for i in range(16):
                    r = base_row + i
                    prods = [_rows[r, pl.ds(kk * 16, 16)] * _wv[kk]
                             for kk in range(D // 16)]
                    while len(prods) > 1:
                        prods = [prods[m] + prods[m + 1]
                                 for m in range(0, len(prods), 2)]
                    s = jnp.full((16,), jnp.sum(prods[0]), dtype=jnp.float32)
                    terms.append(jnp.where(lane_masks[i], s, zero))
                while len(terms) > 1:
                    terms = [terms[m] + terms[m + 1]
                             for m in range(0, len(terms), 2)]
                acc = terms[0]
                sl = pl.ds(_c * CHUNK + g * 16, 16)
                if _k == 0:
                    out_v[sl] = acc + bvec
                else:
                    out_v[sl] = out_v[sl] + acc
                return 0

            lax.fori_loop(0, CHUNK // 16, group_body, 0)
            if t + nbuf < n_chunks:
                copies[t + nbuf] = pltpu.async_copy(
                    tables[(t + nbuf) % 3].at[idx_v.at[t + nbuf]],
                    bufs[t % nbuf], sems[t % nbuf])

        pltpu.sync_copy(out_v, out_hbm.at[pl.ds(base, b_per_w)])

    return sck


def kernel(f1, f2, f3, table1, table2, W, b):
    B = f1.shape[0]
    V, D = table1.shape
    b_per_w = B // NW
    n_bchunk = b_per_w // CHUNK

    f1 = f1.astype(jnp.int32)
    f2 = f2.astype(jnp.int32)
    f3 = f3.astype(jnp.int32)
    idx = jnp.stack([f1, f2, f3], axis=0).reshape(3, NW, n_bchunk, CHUNK)
    idx_all = idx.transpose(1, 2, 0, 3).reshape(NW, 3 * n_bchunk, CHUNK)

    w_flat = W.reshape(-1).astype(jnp.float32)
    b16 = jnp.broadcast_to(b.astype(jnp.float32), (16,))

    sck = _make_sc_kernel(B // 2, D, V, num_cores=1)
    out0 = sck(w_flat, b16, idx_all[:NS], table1, table2)
    out1 = sck(w_flat, b16, idx_all[NS:], table1, table2)
    return jnp.concatenate([out0, out1]).reshape(B, 1)


# speedup vs baseline: 1.4332x; 1.4332x over previous
import functools

import jax
import jax.numpy as jnp
from jax import lax
from jax.experimental import pallas as pl
from jax.experimental.pallas import tpu as pltpu
from jax.experimental.pallas import tpu_sc as plsc

NC = 2
NS = 16
NW = NC * NS
CHUNK = 128


def _make_sc_kernel(B, D, V):
    b_per_w = B // NW
    n_bchunk = b_per_w // CHUNK
    n_chunks = n_bchunk * 3
    mesh = plsc.VectorSubcoreMesh(core_axis_name="c", subcore_axis_name="s")

    @functools.partial(
        pl.kernel,
        mesh=mesh,
        compiler_params=pltpu.CompilerParams(needs_layout_passes=False),
        out_type=jax.ShapeDtypeStruct((B,), jnp.float32),
        scratch_types=[
            pltpu.VMEM((n_chunks, CHUNK), jnp.int32),
            pltpu.VMEM((CHUNK, D), jnp.float32),
            pltpu.VMEM((CHUNK, D), jnp.float32),
            pltpu.VMEM((b_per_w,), jnp.float32),
            pltpu.VMEM((3 * D,), jnp.float32),
            pltpu.VMEM((16,), jnp.float32),
            pltpu.SemaphoreType.DMA,
            pltpu.SemaphoreType.DMA,
        ],
    )
    def sck(w_hbm, b_hbm, f1_hbm, f2_hbm, f3_hbm, t1_hbm, t2_hbm, out_hbm,
            idx_v, rows0, rows1, out_v, w_v, b_v, sem0, sem1):
        cid = lax.axis_index("c")
        sid = lax.axis_index("s")
        wid = cid * NS + sid
        base = wid * b_per_w
        bsl = pl.ds(base, b_per_w)
        feats = (f1_hbm, f2_hbm, f3_hbm)
        for t in range(n_chunks):
            c, k = t // 3, t % 3
            pltpu.sync_copy(
                feats[k].at[pl.ds(base + c * CHUNK, CHUNK)], idx_v.at[t])
        pltpu.sync_copy(w_hbm, w_v)
        pltpu.sync_copy(b_hbm, b_v)

        bufs = (rows0, rows1)
        sems = (sem0, sem1)
        tables = (t1_hbm, t1_hbm, t2_hbm)
        iota16 = jnp.arange(16, dtype=jnp.int32)
        lane_masks = [iota16 == i for i in range(16)]
        bvec = b_v[...]

        copies = [None] * n_chunks
        copies[0] = pltpu.async_copy(
            tables[0].at[idx_v.at[0]], bufs[0], sems[0])
        for t in range(n_chunks):
            c, k = t // 3, t % 3
            if t + 1 < n_chunks:
                copies[t + 1] = pltpu.async_copy(
                    tables[(t + 1) % 3].at[idx_v.at[t + 1]],
                    bufs[(t + 1) % 2], sems[(t + 1) % 2])
            copies[t].wait()
            rows = bufs[t % 2]
            woff = k * D
            wv = [w_v[pl.ds(woff + 16 * kk, 16)] for kk in range(D // 16)]

            def group_body(g, _, _rows=rows, _wv=wv, _c=c, _k=k):
                base_row = g * 16
                zero = jnp.zeros((16,), jnp.float32)
                terms = []
                for i in range(16):
                    r = base_row + i
                    prods = [_rows[r, pl.ds(kk * 16, 16)] * _wv[kk]
                             for kk in range(D // 16)]
                    while len(prods) > 1:
                        prods = [prods[m] + prods[m + 1]
                                 for m in range(0, len(prods), 2)]
                    s = jnp.full((16,), jnp.sum(prods[0]), dtype=jnp.float32)
                    terms.append(jnp.where(lane_masks[i], s, zero))
                while len(terms) > 1:
                    terms = [terms[m] + terms[m + 1]
                             for m in range(0, len(terms), 2)]
                acc = terms[0]
                sl = pl.ds(_c * CHUNK + g * 16, 16)
                if _k == 0:
                    out_v[sl] = acc + bvec
                else:
                    out_v[sl] = out_v[sl] + acc
                return 0

            lax.fori_loop(0, CHUNK // 16, group_body, 0)

        pltpu.sync_copy(out_v, out_hbm.at[bsl])

    return sck


def kernel(f1, f2, f3, table1, table2, W, b):
    B = f1.shape[0]
    V, D = table1.shape

    f1 = f1.astype(jnp.int32)
    f2 = f2.astype(jnp.int32)
    f3 = f3.astype(jnp.int32)
    w_flat = W.reshape(-1).astype(jnp.float32)
    b16 = jnp.broadcast_to(b.astype(jnp.float32), (16,))

    sck = _make_sc_kernel(B, D, V)
    out = sck(w_flat, b16, f1, f2, f3, table1, table2)
    return out.reshape(B, 1)


# speedup vs baseline: 1.5975x; 1.1146x over previous
import functools

import jax
import jax.numpy as jnp
from jax import lax
from jax.experimental import pallas as pl
from jax.experimental.pallas import tpu as pltpu
from jax.experimental.pallas import tpu_sc as plsc

NC = 2
NS = 16
NW = NC * NS
CHUNK = 128


def _make_sc_kernel(B, D, V):
    b_per_w = B // NW
    n_bchunk = b_per_w // CHUNK
    n_chunks = n_bchunk * 3
    mesh = plsc.VectorSubcoreMesh(core_axis_name="c", subcore_axis_name="s")

    @functools.partial(
        pl.kernel,
        mesh=mesh,
        compiler_params=pltpu.CompilerParams(needs_layout_passes=False),
        out_type=jax.ShapeDtypeStruct((B,), jnp.float32),
        scratch_types=[
            pltpu.VMEM((n_chunks, CHUNK), jnp.int32),
            pltpu.VMEM((CHUNK, D), jnp.float32),
            pltpu.VMEM((CHUNK, D), jnp.float32),
            pltpu.VMEM((b_per_w,), jnp.float32),
            pltpu.VMEM((3 * D,), jnp.float32),
            pltpu.VMEM((16,), jnp.float32),
            pltpu.SemaphoreType.DMA,
            pltpu.SemaphoreType.DMA,
        ],
    )
    def sck(w_hbm, b_hbm, idx_hbm, t1_hbm, t2_hbm, out_hbm,
            idx_v, rows0, rows1, out_v, w_v, b_v, sem0, sem1):
        cid = lax.axis_index("c")
        sid = lax.axis_index("s")
        wid = cid * NS + sid
        base = wid * b_per_w
        pltpu.sync_copy(idx_hbm.at[wid], idx_v)
        pltpu.sync_copy(w_hbm, w_v)
        pltpu.sync_copy(b_hbm, b_v)

        bufs = (rows0, rows1)
        sems = (sem0, sem1)
        nbuf = len(bufs)
        tables = (t1_hbm, t1_hbm, t2_hbm)
        iota16 = jnp.arange(16, dtype=jnp.int32)
        lane_masks = [iota16 == i for i in range(16)]
        bvec = b_v[...]

        copies = [None] * n_chunks
        for t in range(min(nbuf, n_chunks)):
            copies[t] = pltpu.async_copy(
                tables[t % 3].at[idx_v.at[t]], bufs[t], sems[t])
        for t in range(n_chunks):
            c, k = t // 3, t % 3
            copies[t].wait()
            rows = bufs[t % nbuf]
            woff = k * D
            wv = [w_v[pl.ds(woff + 16 * kk, 16)] for kk in range(D // 16)]

            def group_body(g, _, _rows=rows, _wv=wv, _c=c, _k=k):
                base_row = g * 16
                zero = jnp.zeros((16,), jnp.float32)
                terms = []
                for i in range(16):
                    r = base_row + i
                    prods = [_rows[r, pl.ds(kk * 16, 16)] * _wv[kk]
                             for kk in range(D // 16)]
                    while len(prods) > 1:
                        prods = [prods[m] + prods[m + 1]
                                 for m in range(0, len(prods), 2)]
                    s = jnp.full((16,), jnp.sum(prods[0]), dtype=jnp.float32)
                    terms.append(jnp.where(lane_masks[i], s, zero))
                while len(terms) > 1:
                    terms = [terms[m] + terms[m + 1]
                             for m in range(0, len(terms), 2)]
                acc = terms[0]
                sl = pl.ds(_c * CHUNK + g * 16, 16)
                if _k == 0:
                    out_v[sl] = acc + bvec
                else:
                    out_v[sl] = out_v[sl] + acc
                return 0

            lax.fori_loop(0, CHUNK // 16, group_body, 0)
            if t + nbuf < n_chunks:
                copies[t + nbuf] = pltpu.async_copy(
                    tables[(t + nbuf) % 3].at[idx_v.at[t + nbuf]],
                    bufs[t % nbuf], sems[t % nbuf])

        pltpu.sync_copy(out_v, out_hbm.at[pl.ds(base, b_per_w)])

    return sck


def kernel(f1, f2, f3, table1, table2, W, b):
    B = f1.shape[0]
    V, D = table1.shape
    b_per_w = B // NW
    n_bchunk = b_per_w // CHUNK

    f1 = f1.astype(jnp.int32)
    f2 = f2.astype(jnp.int32)
    f3 = f3.astype(jnp.int32)
    idx = jnp.stack([f1, f2, f3], axis=0).reshape(3, NW, n_bchunk, CHUNK)
    idx_all = idx.transpose(1, 2, 0, 3).reshape(NW, 3 * n_bchunk, CHUNK)

    w_flat = W.reshape(-1).astype(jnp.float32)
    b16 = jnp.broadcast_to(b.astype(jnp.float32), (16,))

    sck = _make_sc_kernel(B, D, V)
    out = sck(w_flat, b16, idx_all, table1, table2)
    return out.reshape(B, 1)
